# baseline (speedup 1.0000x reference)
import functools

import jax
import jax.numpy as jnp
from jax import lax
from jax.experimental import pallas as pl
from jax.experimental.pallas import tpu as pltpu
from jax.experimental.pallas import tpu_sc as plsc

_NC = 2
_NS = 16
_NW = _NC * _NS
_L = 16

_N = 2048 * 2048
_PER_W = _N // _NW
_CHUNK = 16384
_NCHUNKS = _PER_W // _CHUNK


def _remap_sc(label_flat, mapping):
    mesh = plsc.VectorSubcoreMesh(core_axis_name="c", subcore_axis_name="s")

    @functools.partial(
        pl.kernel,
        out_type=jax.ShapeDtypeStruct((_N,), jnp.int32),
        mesh=mesh,
        compiler_params=pltpu.CompilerParams(needs_layout_passes=False),
        scratch_types=[
            pltpu.VMEM((128,), jnp.int32),
            pltpu.VMEM((_CHUNK,), jnp.int32),
            pltpu.VMEM((_CHUNK,), jnp.int32),
        ],
    )
    def k(label_hbm, map_hbm, out_hbm, table_v, lab_v, res_v):
        wid = lax.axis_index("s") * _NC + lax.axis_index("c")
        pltpu.sync_copy(map_hbm, table_v.at[pl.ds(0, 34)])

        for ci in range(_NCHUNKS):
            base = wid * _PER_W + ci * _CHUNK
            pltpu.sync_copy(label_hbm.at[pl.ds(base, _CHUNK)], lab_v)

            def body(i, _):
                idx = lab_v[pl.ds(i * _L, _L)]
                res_v[pl.ds(i * _L, _L)] = plsc.load_gather(table_v, [idx])
                return 0

            lax.fori_loop(0, _CHUNK // _L, body, 0)
            pltpu.sync_copy(res_v, out_hbm.at[pl.ds(base, _CHUNK)])

    return k(label_flat, mapping)


def kernel(image, label, mapping):
    label_flat = label.reshape(-1).astype(jnp.int32)
    remapped = _remap_sc(label_flat, mapping.astype(jnp.int32))
    return (image, remapped.reshape(label.shape).astype(mapping.dtype))


# speedup vs baseline: 1.4990x; 1.4990x over previous
import functools

import jax
import jax.numpy as jnp
from jax import lax
from jax.experimental import pallas as pl
from jax.experimental.pallas import tpu as pltpu
from jax.experimental.pallas import tpu_sc as plsc

_NC = 2
_NS = 16
_NW = _NC * _NS
_L = 16

_N = 2048 * 2048
_PER_W = _N // _NW
_CHUNK = 16384
_NCHUNKS = _PER_W // _CHUNK


def _remap_sc(label_flat, mapping):
    mesh = plsc.VectorSubcoreMesh(core_axis_name="c", subcore_axis_name="s")

    @functools.partial(
        pl.kernel,
        out_type=jax.ShapeDtypeStruct((_N,), jnp.int32),
        mesh=mesh,
        compiler_params=pltpu.CompilerParams(needs_layout_passes=False),
        scratch_types=[
            pltpu.VMEM((128,), jnp.int32),
            pltpu.VMEM((_CHUNK,), jnp.int32),
            pltpu.VMEM((_CHUNK,), jnp.int32),
            pltpu.VMEM((_CHUNK,), jnp.int32),
            pltpu.VMEM((_CHUNK,), jnp.int32),
            pltpu.SemaphoreType.DMA,
            pltpu.SemaphoreType.DMA,
            pltpu.SemaphoreType.DMA,
            pltpu.SemaphoreType.DMA,
        ],
    )
    def k(label_hbm, map_hbm, out_hbm, table_v,
          lab0, lab1, res0, res1, isem0, isem1, osem0, osem1):
        wid = lax.axis_index("s") * _NC + lax.axis_index("c")
        labs = (lab0, lab1)
        ress = (res0, res1)
        isems = (isem0, isem1)
        osems = (osem0, osem1)

        pltpu.sync_copy(map_hbm, table_v.at[pl.ds(0, 34)])

        def in_copy(ci):
            base = wid * _PER_W + ci * _CHUNK
            return pltpu.make_async_copy(
                label_hbm.at[pl.ds(base, _CHUNK)], labs[ci % 2], isems[ci % 2])

        def out_copy(ci):
            base = wid * _PER_W + ci * _CHUNK
            return pltpu.make_async_copy(
                ress[ci % 2], out_hbm.at[pl.ds(base, _CHUNK)], osems[ci % 2])

        in_copy(0).start()
        in_copy(1).start()

        for ci in range(_NCHUNKS):
            b = ci % 2
            in_copy(ci).wait()
            if ci >= 2:
                out_copy(ci - 2).wait()

            lab_v, res_v = labs[b], ress[b]

            @plsc.parallel_loop(0, _CHUNK, _L, unroll=8)
            def _(i):
                idx = lab_v[pl.ds(i, _L)]
                res_v[pl.ds(i, _L)] = plsc.load_gather(table_v, [idx])

            out_copy(ci).start()
            if ci + 2 < _NCHUNKS:
                in_copy(ci + 2).start()

        out_copy(_NCHUNKS - 2).wait()
        out_copy(_NCHUNKS - 1).wait()

    return k(label_flat, mapping)


def kernel(image, label, mapping):
    label_flat = label.reshape(-1).astype(jnp.int32)
    remapped = _remap_sc(label_flat, mapping.astype(jnp.int32))
    return (image, remapped.reshape(label.shape).astype(mapping.dtype))


# speedup vs baseline: 2.1229x; 1.4163x over previous
import functools

import jax
import jax.numpy as jnp
from jax import lax
from jax.experimental import pallas as pl
from jax.experimental.pallas import tpu as pltpu
from jax.experimental.pallas import tpu_sc as plsc

_NC = 2
_NS = 16
_NW = _NC * _NS
_L = 16

_ROWS = 2048
_COLS = 2048
_ROWS_PER_W = _ROWS // _NW
_CR = 8
_NCHUNKS = _ROWS_PER_W // _CR


def _remap_sc(label, mapping):
    mesh = plsc.VectorSubcoreMesh(core_axis_name="c", subcore_axis_name="s")

    @functools.partial(
        pl.kernel,
        out_type=jax.ShapeDtypeStruct((_ROWS, _COLS), jnp.int32),
        mesh=mesh,
        compiler_params=pltpu.CompilerParams(needs_layout_passes=False),
        scratch_types=[
            pltpu.VMEM((128,), jnp.int32),
            pltpu.VMEM((_CR, _COLS), jnp.int32),
            pltpu.VMEM((_CR, _COLS), jnp.int32),
            pltpu.VMEM((_CR, _COLS), jnp.int32),
            pltpu.VMEM((_CR, _COLS), jnp.int32),
            pltpu.SemaphoreType.DMA,
            pltpu.SemaphoreType.DMA,
            pltpu.SemaphoreType.DMA,
            pltpu.SemaphoreType.DMA,
        ],
    )
    def k(label_hbm, map_hbm, out_hbm, table_v,
          lab0, lab1, res0, res1, isem0, isem1, osem0, osem1):
        wid = lax.axis_index("s") * _NC + lax.axis_index("c")
        labs = (lab0, lab1)
        ress = (res0, res1)
        isems = (isem0, isem1)
        osems = (osem0, osem1)

        pltpu.sync_copy(map_hbm, table_v.at[pl.ds(0, 34)])

        def in_copy(ci):
            row0 = wid * _ROWS_PER_W + ci * _CR
            return pltpu.make_async_copy(
                label_hbm.at[pl.ds(row0, _CR), :], labs[ci % 2], isems[ci % 2])

        def out_copy(ci):
            row0 = wid * _ROWS_PER_W + ci * _CR
            return pltpu.make_async_copy(
                ress[ci % 2], out_hbm.at[pl.ds(row0, _CR), :], osems[ci % 2])

        in_copy(0).start()
        in_copy(1).start()

        for ci in range(_NCHUNKS):
            b = ci % 2
            in_copy(ci).wait()
            if ci >= 2:
                out_copy(ci - 2).wait()

            lab_v, res_v = labs[b], ress[b]

            for r in range(_CR):
                @plsc.parallel_loop(0, _COLS, _L, unroll=8)
                def _(i):
                    idx = lab_v[r, pl.ds(i, _L)]
                    res_v[r, pl.ds(i, _L)] = plsc.load_gather(table_v, [idx])

            out_copy(ci).start()
            if ci + 2 < _NCHUNKS:
                in_copy(ci + 2).start()

        out_copy(_NCHUNKS - 2).wait()
        out_copy(_NCHUNKS - 1).wait()

    return k(label, mapping)


def kernel(image, label, mapping):
    remapped = _remap_sc(label.astype(jnp.int32), mapping.astype(jnp.int32))
    return (image, remapped.astype(mapping.dtype))


# speedup vs baseline: 2.6039x; 1.2265x over previous
import functools

import jax
import jax.numpy as jnp
from jax import lax
from jax.experimental import pallas as pl
from jax.experimental.pallas import tpu as pltpu
from jax.experimental.pallas import tpu_sc as plsc

_NC = 2
_NS = 16
_NW = _NC * _NS
_L = 16

_ROWS = 2048
_COLS = 2048
_ROWS_PER_W = _ROWS // _NW
_CR = 8
_NCHUNKS = _ROWS_PER_W // _CR


def _remap_sc(label, mapping):
    mesh = plsc.VectorSubcoreMesh(core_axis_name="c", subcore_axis_name="s")

    @functools.partial(
        pl.kernel,
        out_type=jax.ShapeDtypeStruct((_ROWS, _COLS), jnp.int32),
        mesh=mesh,
        compiler_params=pltpu.CompilerParams(needs_layout_passes=False),
        scratch_types=[
            pltpu.VMEM((128,), jnp.int32),
            pltpu.VMEM((_CR, _COLS), jnp.int32),
            pltpu.VMEM((_CR, _COLS), jnp.int32),
            pltpu.VMEM((_CR, _COLS), jnp.int32),
            pltpu.VMEM((_CR, _COLS), jnp.int32),
            pltpu.SemaphoreType.DMA,
            pltpu.SemaphoreType.DMA,
            pltpu.SemaphoreType.DMA,
            pltpu.SemaphoreType.DMA,
        ],
    )
    def k(label_hbm, map_hbm, out_hbm, table_v,
          lab0, lab1, res0, res1, isem0, isem1, osem0, osem1):
        wid = lax.axis_index("s") * _NC + lax.axis_index("c")
        labs = (lab0, lab1)
        ress = (res0, res1)
        isems = (isem0, isem1)
        osems = (osem0, osem1)

        pltpu.sync_copy(map_hbm, table_v.at[pl.ds(0, 34)])

        def in_copy(ci):
            row0 = wid * _ROWS_PER_W + ci * _CR
            return pltpu.make_async_copy(
                label_hbm.at[pl.ds(row0, _CR), :], labs[ci % 2], isems[ci % 2])

        def out_copy(ci):
            row0 = wid * _ROWS_PER_W + ci * _CR
            return pltpu.make_async_copy(
                ress[ci % 2], out_hbm.at[pl.ds(row0, _CR), :], osems[ci % 2])

        in_copy(0).start()
        in_copy(1).start()

        for ci in range(_NCHUNKS):
            b = ci % 2
            in_copy(ci).wait()
            if ci >= 2:
                out_copy(ci - 2).wait()

            lab_v, res_v = labs[b], ress[b]

            for r in range(_CR):
                @plsc.parallel_loop(0, _COLS, _L, unroll=8)
                def _(i):
                    idx = lab_v[r, pl.ds(i, _L)]
                    res_v[r, pl.ds(i, _L)] = plsc.load_gather(table_v, [idx])

            out_copy(ci).start()
            if ci + 2 < _NCHUNKS:
                in_copy(ci + 2).start()

        out_copy(_NCHUNKS - 2).wait()
        out_copy(_NCHUNKS - 1).wait()

    return k(label, mapping)


def _copy_body(x_ref, o_ref):
    o_ref[...] = x_ref[...]


def _image_copy_tc(image):
    c, h, w = image.shape
    bh = 512
    return pl.pallas_call(
        _copy_body,
        grid=(c, h // bh),
        in_specs=[pl.BlockSpec((1, bh, w), lambda i, j: (i, j, 0))],
        out_specs=pl.BlockSpec((1, bh, w), lambda i, j: (i, j, 0)),
        out_shape=jax.ShapeDtypeStruct(image.shape, image.dtype),
    )(image)


def kernel(image, label, mapping):
    remapped = _remap_sc(label.astype(jnp.int32), mapping.astype(jnp.int32))
    return (_image_copy_tc(image), remapped.astype(mapping.dtype))


# speedup vs baseline: 2.6494x; 1.0175x over previous
import functools

import jax
import jax.numpy as jnp
from jax import lax
from jax.experimental import pallas as pl
from jax.experimental.pallas import tpu as pltpu
from jax.experimental.pallas import tpu_sc as plsc

_NC = 2
_NS = 16
_NW = _NC * _NS
_L = 16

_ROWS = 2048
_COLS = 2048
_ROWS_PER_W = _ROWS // _NW
_CR = 8
_NCHUNKS = _ROWS_PER_W // _CR


def _remap_sc(label, mapping):
    mesh = plsc.VectorSubcoreMesh(core_axis_name="c", subcore_axis_name="s")

    @functools.partial(
        pl.kernel,
        out_type=jax.ShapeDtypeStruct((_ROWS, _COLS), jnp.int32),
        mesh=mesh,
        compiler_params=pltpu.CompilerParams(needs_layout_passes=False),
        scratch_types=[
            pltpu.VMEM((128,), jnp.int32),
            pltpu.VMEM((_CR, _COLS), jnp.int32),
            pltpu.VMEM((_CR, _COLS), jnp.int32),
            pltpu.VMEM((_CR, _COLS), jnp.int32),
            pltpu.VMEM((_CR, _COLS), jnp.int32),
            pltpu.SemaphoreType.DMA,
            pltpu.SemaphoreType.DMA,
            pltpu.SemaphoreType.DMA,
            pltpu.SemaphoreType.DMA,
        ],
    )
    def k(label_hbm, map_hbm, out_hbm, table_v,
          lab0, lab1, res0, res1, isem0, isem1, osem0, osem1):
        wid = lax.axis_index("s") * _NC + lax.axis_index("c")
        labs = (lab0, lab1)
        ress = (res0, res1)
        isems = (isem0, isem1)
        osems = (osem0, osem1)

        pltpu.sync_copy(map_hbm, table_v.at[pl.ds(0, 34)])

        def in_copy(ci):
            row0 = wid * _ROWS_PER_W + ci * _CR
            return pltpu.make_async_copy(
                label_hbm.at[pl.ds(row0, _CR), :], labs[ci % 2], isems[ci % 2])

        def out_copy(ci):
            row0 = wid * _ROWS_PER_W + ci * _CR
            return pltpu.make_async_copy(
                ress[ci % 2], out_hbm.at[pl.ds(row0, _CR), :], osems[ci % 2])

        in_copy(0).start()
        in_copy(1).start()

        for ci in range(_NCHUNKS):
            b = ci % 2
            in_copy(ci).wait()
            if ci >= 2:
                out_copy(ci - 2).wait()

            lab_v, res_v = labs[b], ress[b]

            for r in range(_CR):
                @plsc.parallel_loop(0, _COLS, _L, unroll=8)
                def _(i):
                    idx = lab_v[r, pl.ds(i, _L)]
                    res_v[r, pl.ds(i, _L)] = plsc.load_gather(table_v, [idx])

            out_copy(ci).start()
            if ci + 2 < _NCHUNKS:
                in_copy(ci + 2).start()

        out_copy(_NCHUNKS - 2).wait()
        out_copy(_NCHUNKS - 1).wait()

    return k(label, mapping)


def _copy_body(x_ref, o_ref):
    o_ref[...] = x_ref[...]


def _image_copy_tc(image):
    c, h, w = image.shape
    bh = 1024
    return pl.pallas_call(
        _copy_body,
        grid=(c, h // bh),
        in_specs=[pl.BlockSpec((1, bh, w), lambda i, j: (i, j, 0))],
        out_specs=pl.BlockSpec((1, bh, w), lambda i, j: (i, j, 0)),
        out_shape=jax.ShapeDtypeStruct(image.shape, image.dtype),
    )(image)


def kernel(image, label, mapping):
    remapped = _remap_sc(label.astype(jnp.int32), mapping.astype(jnp.int32))
    return (_image_copy_tc(image), remapped.astype(mapping.dtype))


# speedup vs baseline: 2.6532x; 1.0014x over previous
import functools

import jax
import jax.numpy as jnp
from jax import lax
from jax.experimental import pallas as pl
from jax.experimental.pallas import tpu as pltpu
from jax.experimental.pallas import tpu_sc as plsc

_NC = 2
_NS = 16
_NW = _NC * _NS
_L = 16

_ROWS = 2048
_COLS = 2048
_ROWS_PER_W = _ROWS // _NW
_CR = 8
_NCHUNKS = _ROWS_PER_W // _CR


def _remap_sc(label, mapping):
    mesh = plsc.VectorSubcoreMesh(core_axis_name="c", subcore_axis_name="s")

    @functools.partial(
        pl.kernel,
        out_type=jax.ShapeDtypeStruct((_ROWS, _COLS), jnp.int32),
        mesh=mesh,
        compiler_params=pltpu.CompilerParams(needs_layout_passes=False),
        scratch_types=[
            pltpu.VMEM((128,), jnp.int32),
            pltpu.VMEM((_CR, _COLS), jnp.int32),
            pltpu.VMEM((_CR, _COLS), jnp.int32),
            pltpu.VMEM((_CR, _COLS), jnp.int32),
            pltpu.VMEM((_CR, _COLS), jnp.int32),
            pltpu.SemaphoreType.DMA,
            pltpu.SemaphoreType.DMA,
            pltpu.SemaphoreType.DMA,
            pltpu.SemaphoreType.DMA,
        ],
    )
    def k(label_hbm, map_hbm, out_hbm, table_v,
          lab0, lab1, res0, res1, isem0, isem1, osem0, osem1):
        wid = lax.axis_index("s") * _NC + lax.axis_index("c")
        labs = (lab0, lab1)
        ress = (res0, res1)
        isems = (isem0, isem1)
        osems = (osem0, osem1)

        pltpu.sync_copy(map_hbm, table_v.at[pl.ds(0, 34)])

        def in_copy(ci):
            row0 = wid * _ROWS_PER_W + ci * _CR
            return pltpu.make_async_copy(
                label_hbm.at[pl.ds(row0, _CR), :], labs[ci % 2], isems[ci % 2])

        def out_copy(ci):
            row0 = wid * _ROWS_PER_W + ci * _CR
            return pltpu.make_async_copy(
                ress[ci % 2], out_hbm.at[pl.ds(row0, _CR), :], osems[ci % 2])

        in_copy(0).start()
        in_copy(1).start()

        for ci in range(_NCHUNKS):
            b = ci % 2
            in_copy(ci).wait()
            if ci >= 2:
                out_copy(ci - 2).wait()

            lab_v, res_v = labs[b], ress[b]

            for r in range(_CR):
                @plsc.parallel_loop(0, _COLS, _L, unroll=8)
                def _(i):
                    idx = lab_v[r, pl.ds(i, _L)]
                    res_v[r, pl.ds(i, _L)] = plsc.load_gather(table_v, [idx])

            out_copy(ci).start()
            if ci + 2 < _NCHUNKS:
                in_copy(ci + 2).start()

        out_copy(_NCHUNKS - 2).wait()
        out_copy(_NCHUNKS - 1).wait()

    return k(label, mapping)


def _copy_body(x_ref, o_ref):
    o_ref[...] = x_ref[...]


def _image_copy_tc(image):
    c, h, w = image.shape
    bh = 1024
    return pl.pallas_call(
        _copy_body,
        grid=(c, h // bh),
        in_specs=[pl.BlockSpec((1, bh, w), lambda i, j: (i, j, 0))],
        out_specs=pl.BlockSpec((1, bh, w), lambda i, j: (i, j, 0)),
        out_shape=jax.ShapeDtypeStruct(image.shape, image.dtype),
    )(image)


def kernel(image, label, mapping):
    image_out = _image_copy_tc(image)
    remapped = _remap_sc(label.astype(jnp.int32), mapping.astype(jnp.int32))
    return (image_out, remapped.astype(mapping.dtype))


# speedup vs baseline: 2.6692x; 1.0060x over previous
import functools

import jax
import jax.numpy as jnp
from jax import lax
from jax.experimental import pallas as pl
from jax.experimental.pallas import tpu as pltpu
from jax.experimental.pallas import tpu_sc as plsc

_NC = 2
_NS = 16
_NW = _NC * _NS
_L = 16

_ROWS = 2048
_COLS = 2048
_ROWS_PER_W = _ROWS // _NW
_CR = 8
_NCHUNKS = _ROWS_PER_W // _CR


def _remap_sc(label, mapping):
    mesh = plsc.VectorSubcoreMesh(core_axis_name="c", subcore_axis_name="s")

    @functools.partial(
        pl.kernel,
        out_type=jax.ShapeDtypeStruct((_ROWS, _COLS), jnp.int32),
        mesh=mesh,
        compiler_params=pltpu.CompilerParams(needs_layout_passes=False),
        scratch_types=[
            pltpu.VMEM((128,), jnp.int32),
            pltpu.VMEM((_CR, _COLS), jnp.int32),
            pltpu.VMEM((_CR, _COLS), jnp.int32),
            pltpu.VMEM((_CR, _COLS), jnp.int32),
            pltpu.VMEM((_CR, _COLS), jnp.int32),
            pltpu.SemaphoreType.DMA,
            pltpu.SemaphoreType.DMA,
            pltpu.SemaphoreType.DMA,
            pltpu.SemaphoreType.DMA,
        ],
    )
    def k(label_hbm, map_hbm, out_hbm, table_v,
          lab0, lab1, res0, res1, isem0, isem1, osem0, osem1):
        wid = lax.axis_index("s") * _NC + lax.axis_index("c")
        labs = (lab0, lab1)
        ress = (res0, res1)
        isems = (isem0, isem1)
        osems = (osem0, osem1)

        pltpu.sync_copy(map_hbm, table_v.at[pl.ds(0, 34)])

        def in_copy(ci, b):
            row0 = wid * _ROWS_PER_W + ci * _CR
            return pltpu.make_async_copy(
                label_hbm.at[pl.ds(row0, _CR), :], labs[b], isems[b])

        def out_copy(ci, b):
            row0 = wid * _ROWS_PER_W + ci * _CR
            return pltpu.make_async_copy(
                ress[b], out_hbm.at[pl.ds(row0, _CR), :], osems[b])

        def gather_chunk(b):
            lab_v, res_v = labs[b], ress[b]
            for r in range(_CR):
                @plsc.parallel_loop(0, _COLS, _L, unroll=8)
                def _(i):
                    idx = lab_v[r, pl.ds(i, _L)]
                    res_v[r, pl.ds(i, _L)] = plsc.load_gather(table_v, [idx])

        in_copy(0, 0).start()
        in_copy(1, 1).start()

        def pair_body(ci2, _):
            for b in (0, 1):
                ci = 2 * ci2 + b
                in_copy(ci, b).wait()

                @pl.when(ci >= 2)
                def _():
                    out_copy(ci - 2, b).wait()

                gather_chunk(b)
                out_copy(ci, b).start()

                @pl.when(ci + 2 < _NCHUNKS)
                def _():
                    in_copy(ci + 2, b).start()
            return 0

        lax.fori_loop(0, _NCHUNKS // 2, pair_body, 0)

        out_copy(_NCHUNKS - 2, 0).wait()
        out_copy(_NCHUNKS - 1, 1).wait()

    return k(label, mapping)


def _copy_body(x_ref, o_ref):
    o_ref[...] = x_ref[...]


def _image_copy_tc(image):
    c, h, w = image.shape
    bh = 1024
    return pl.pallas_call(
        _copy_body,
        grid=(c, h // bh),
        in_specs=[pl.BlockSpec((1, bh, w), lambda i, j: (i, j, 0))],
        out_specs=pl.BlockSpec((1, bh, w), lambda i, j: (i, j, 0)),
        out_shape=jax.ShapeDtypeStruct(image.shape, image.dtype),
    )(image)


def kernel(image, label, mapping):
    image_out = _image_copy_tc(image)
    remapped = _remap_sc(label.astype(jnp.int32), mapping.astype(jnp.int32))
    return (image_out, remapped.astype(mapping.dtype))
